# baseline (speedup 1.0000x reference)
import jax
import jax.numpy as jnp
from jax.experimental import pallas as pl
from jax.experimental.pallas import tpu as pltpu

_TEMP = 0.05
_B = 1024
_D = 64
_N = 100000
_BN = 10000
_GRID = _N // _BN
_CHUNK = _B // 8
_LOG2E = 1.4426950408889634
_LN2 = 0.6931471805599453
_C2 = 84.0


def _ce_kernel(t_sm, x_ref, f_ref, feat_any, out_ref, mc_ref, s_ref,
               rows_ref, row_sem):
    i = pl.program_id(0)

    @pl.when(i == 0)
    def _init():
        x2 = x_ref[...]
        m2 = jnp.sqrt(jnp.sum(x2 * x2, axis=1, keepdims=True))
        mc_ref[...] = m2 - _C2
        s_ref[...] = jnp.zeros_like(s_ref)

    @pl.when(i < 8)
    def _issue_gathers():
        base = i * _CHUNK

        def _issue(j, carry):
            pltpu.make_async_copy(
                feat_any.at[pl.ds(t_sm[base + j], 1), :],
                rows_ref.at[pl.ds(base + j, 1), :],
                row_sem,
            ).start()
            return carry

        jax.lax.fori_loop(0, _CHUNK, _issue, 0)

    z = jax.lax.dot_general(
        x_ref[...].astype(jnp.bfloat16), f_ref[...],
        (((1,), (1,)), ((), ())),
        preferred_element_type=jnp.float32,
    )
    e = jnp.exp2(z - mc_ref[...])
    s_ref[...] += jnp.sum(e, axis=1, keepdims=True)

    @pl.when(i == _GRID - 1)
    def _fin():
        pltpu.make_async_copy(
            feat_any.at[pl.ds(0, _B), :], rows_ref, row_sem
        ).wait()
        tgt2 = jnp.sum(x_ref[...] * rows_ref[...], axis=1, keepdims=True)
        lse2 = mc_ref[...] + jnp.log2(s_ref[...])
        out_ref[...] = jnp.sum(lse2 - tgt2, keepdims=True) * (_LN2 / _B)


def kernel(inputs, features, targets):
    x = inputs * (_LOG2E / _TEMP)
    fb = features.astype(jnp.bfloat16)
    t = targets.astype(jnp.int32)
    grid_spec = pltpu.PrefetchScalarGridSpec(
        num_scalar_prefetch=1,
        grid=(_GRID,),
        in_specs=[
            pl.BlockSpec((_B, _D), lambda i, *_: (0, 0)),
            pl.BlockSpec((_BN, _D), lambda i, *_: (i, 0)),
            pl.BlockSpec(memory_space=pltpu.MemorySpace.HBM),
        ],
        out_specs=pl.BlockSpec((1, 1), lambda i, *_: (0, 0)),
        scratch_shapes=[
            pltpu.VMEM((_B, 1), jnp.float32),
            pltpu.VMEM((_B, 1), jnp.float32),
            pltpu.VMEM((_B, _D), jnp.float32),
            pltpu.SemaphoreType.DMA,
        ],
    )
    out = pl.pallas_call(
        _ce_kernel,
        grid_spec=grid_spec,
        out_shape=jax.ShapeDtypeStruct((1, 1), jnp.float32),
    )(t, x, fb, features)
    return out[0, 0]


# speedup vs baseline: 1.0338x; 1.0338x over previous
import jax
import jax.numpy as jnp
from jax.experimental import pallas as pl
from jax.experimental.pallas import tpu as pltpu

_TEMP = 0.05
_B = 1024
_D = 64
_N = 100000
_BN = 10000
_GRID = _N // _BN
_LOG2E = 1.4426950408889634
_LN2 = 0.6931471805599453
_C2 = 84.0


def _ce_kernel(x_ref, f_ref, t_ref, out_ref, mc_ref, s_ref, g_ref):
    i = pl.program_id(0)

    @pl.when(i == 0)
    def _init():
        x2 = x_ref[...]
        m2 = jnp.sqrt(jnp.sum(x2 * x2, axis=1, keepdims=True))
        mc_ref[...] = m2 - _C2
        s_ref[...] = jnp.zeros_like(s_ref)
        g_ref[...] = jnp.zeros_like(g_ref)

    z = jax.lax.dot_general(
        x_ref[...].astype(jnp.bfloat16), f_ref[...],
        (((1,), (1,)), ((), ())),
        preferred_element_type=jnp.float32,
    )
    e = jnp.exp2(z - mc_ref[...])
    s_ref[...] += jnp.sum(e, axis=1, keepdims=True)

    col = jax.lax.broadcasted_iota(jnp.int32, z.shape, 1) + i * _BN
    hit = col == t_ref[...]
    g_ref[...] += jnp.sum(jnp.where(hit, z, 0.0), axis=1, keepdims=True)

    @pl.when(i == _GRID - 1)
    def _fin():
        lse2 = mc_ref[...] + jnp.log2(s_ref[...])
        out_ref[...] = jnp.sum(lse2 - g_ref[...], keepdims=True) * (_LN2 / _B)


def kernel(inputs, features, targets):
    x = inputs * (_LOG2E / _TEMP)
    fb = features.astype(jnp.bfloat16)
    t = targets.astype(jnp.int32).reshape(_B, 1)
    out = pl.pallas_call(
        _ce_kernel,
        grid=(_GRID,),
        in_specs=[
            pl.BlockSpec((_B, _D), lambda i: (0, 0)),
            pl.BlockSpec((_BN, _D), lambda i: (i, 0)),
            pl.BlockSpec((_B, 1), lambda i: (0, 0)),
        ],
        out_specs=pl.BlockSpec((1, 1), lambda i: (0, 0)),
        out_shape=jax.ShapeDtypeStruct((1, 1), jnp.float32),
        scratch_shapes=[
            pltpu.VMEM((_B, 1), jnp.float32),
            pltpu.VMEM((_B, 1), jnp.float32),
            pltpu.VMEM((_B, 1), jnp.float32),
        ],
    )(x, fb, t)
    return out[0, 0]
